# baseline (speedup 1.0000x reference)
import functools

import jax
import jax.numpy as jnp
from jax import lax
from jax.experimental import pallas as pl
from jax.experimental.pallas import tpu as pltpu
from jax.experimental.pallas import tpu_sc as plsc

NC = 2
NS = 16
L = 16
NW = NC * NS

D = 64
CH = D // L
B = 16384
RPW = B // NW
GPW = RPW // L
A_PAD = 1024
APW = A_PAD // NW


def _dot16(rows_ref, base, cc):
    lane = lax.iota(jnp.int32, L)
    acc = jnp.zeros((L,), jnp.float32)
    for i in range(L):
        r = base + i
        p = rows_ref[r, pl.ds(0, L)] * cc[0]
        for c in range(1, len(cc)):
            p = p + rows_ref[r, pl.ds(c * L, L)] * cc[c]
        acc = jnp.where(lane == i, jnp.sum(p), acc)
    return acc


def _sc_body(table, atable, w1t, b1, w2t, b2, idx2, act2, mlp_idx,
             out_logit, out_act,
             idx_v, rows_v, char_v, mlpi_v, mlprow_v,
             w1t_v, b1_v, w2t_v, b2_v,
             aidx_v, arows_v, lgt_v, aout_v,
             sem_rows, sem_mlp, sem_act):
    w = lax.axis_index("s") * NC + lax.axis_index("c")

    pltpu.sync_copy(idx2.at[pl.ds(w * 4, 4)], idx_v)
    row_cps = [
        pltpu.async_copy(
            table.at[idx_v.at[j]], rows_v.at[pl.ds(j * 128, 128)], sem_rows)
        for j in range(4)
    ]

    pltpu.sync_copy(mlp_idx, mlpi_v)
    mlp_cp = pltpu.async_copy(table.at[mlpi_v], mlprow_v, sem_mlp)

    pltpu.sync_copy(act2.at[pl.ds(w, 1)], aidx_v)
    act_cp = pltpu.async_copy(atable.at[aidx_v.at[0]], arows_v, sem_act)

    pltpu.sync_copy(table.at[pl.ds(0, 1)], char_v)
    pltpu.sync_copy(w1t, w1t_v)
    pltpu.sync_copy(b1, b1_v)
    pltpu.sync_copy(w2t, w2t_v)
    pltpu.sync_copy(b2, b2_v)

    cc = [char_v[0, pl.ds(c * L, L)] for c in range(CH)]

    mlp_cp.wait()
    cat = cc + [mlprow_v[0, pl.ds(c * L, L)] for c in range(CH)]
    h = []
    for jg in range(CH):
        sl = pl.ds(jg * L, L)
        h.append(jnp.maximum(_dot16(w1t_v, jg * L, cat) + b1_v[sl], 0.0))
    ec = []
    for jg in range(CH):
        sl = pl.ds(jg * L, L)
        ec.append(_dot16(w2t_v, jg * L, h) + b2_v[sl])

    act_cp.wait()
    for g in range(APW // L):
        aout_v[pl.ds(g * L, L)] = _dot16(arows_v, g * L, ec)
    pltpu.sync_copy(aout_v, out_act.at[pl.ds(w * APW, APW)])

    for cp in row_cps:
        cp.wait()

    def group(g, carry):
        base = pl.multiple_of(g * L, L)
        lgt_v[pl.ds(base, L)] = _dot16(rows_v, base, cc)
        return carry

    lax.fori_loop(0, GPW, group, jnp.int32(0))
    pltpu.sync_copy(lgt_v, out_logit.at[pl.ds(w * RPW, RPW)])


@jax.jit
def _run(object_table, action_table, W1t, b1, W2t, b2, idx2, act2, mlp_idx):
    mesh = plsc.VectorSubcoreMesh(
        core_axis_name="c", subcore_axis_name="s",
        num_cores=NC, num_subcores=NS,
    )
    call = functools.partial(
        pl.kernel,
        out_type=(
            jax.ShapeDtypeStruct((B,), jnp.float32),
            jax.ShapeDtypeStruct((A_PAD,), jnp.float32),
        ),
        mesh=mesh,
        compiler_params=pltpu.CompilerParams(
            needs_layout_passes=False, use_tc_tiling_on_sc=False),
        scratch_types=[
            pltpu.VMEM((4, 128), jnp.int32),
            pltpu.VMEM((RPW, D), jnp.float32),
            pltpu.VMEM((1, D), jnp.float32),
            pltpu.VMEM((8,), jnp.int32),
            pltpu.VMEM((8, D), jnp.float32),
            pltpu.VMEM((D, 2 * D), jnp.float32),
            pltpu.VMEM((D,), jnp.float32),
            pltpu.VMEM((D, D), jnp.float32),
            pltpu.VMEM((D,), jnp.float32),
            pltpu.VMEM((1, APW), jnp.int32),
            pltpu.VMEM((APW, D), jnp.float32),
            pltpu.VMEM((RPW,), jnp.float32),
            pltpu.VMEM((APW,), jnp.float32),
            pltpu.SemaphoreType.DMA,
            pltpu.SemaphoreType.DMA,
            pltpu.SemaphoreType.DMA,
        ],
    )(_sc_body)
    return call(object_table, action_table, W1t, b1, W2t, b2, idx2, act2, mlp_idx)


def kernel(object_table, action_table, W1, b1, W2, b2, node_name_ids, action_ids):
    idx2 = node_name_ids.reshape(B // 128, 128)
    act2 = jnp.concatenate(
        [action_ids, jnp.zeros((A_PAD - action_ids.shape[0],), jnp.int32)]
    ).reshape(NW, APW)
    mlp_idx = node_name_ids[:8]
    out_logit, out_act = _run(
        object_table, action_table, W1.T, b1, W2.T, b2, idx2, act2, mlp_idx)
    return jnp.concatenate([out_logit, out_act[: action_ids.shape[0]]])


# speedup vs baseline: 5.3877x; 5.3877x over previous
import functools

import jax
import jax.numpy as jnp
from jax import lax
from jax.experimental import pallas as pl
from jax.experimental.pallas import tpu as pltpu
from jax.experimental.pallas import tpu_sc as plsc

NC = 2
NS = 16
L = 16
NW = NC * NS

V = 1000000
D = 64
CH = D // L
B = 16384
RPW = B // NW
GPW = RPW // L
A_PAD = 1024
APW = A_PAD // NW

BLK = 32768
NBLK = (V + BLK - 1) // BLK
DROWS = NBLK * (BLK // 128)


def _tc_sweep_body(sref, tt_ref, dots_ref, char_ref, obj_ref, char_sc):
    i = pl.program_id(0)

    @pl.when(i == 0)
    def _():
        char_sc[...] = tt_ref[:, 0:1]
        char_ref[...] = tt_ref[:, 0:1]

    r0 = sref[0]

    @pl.when(i == r0 // BLK)
    def _():
        col = r0 % BLK
        win = tt_ref[:, pl.ds(pl.multiple_of((col // 128) * 128, 128), 128)]
        sel = lax.broadcasted_iota(jnp.int32, (D, 128), 1) == (col % 128)
        obj_ref[...] = jnp.sum(jnp.where(sel, win, 0.0), axis=1, keepdims=True)

    prod = tt_ref[...] * char_sc[...]
    dots = jnp.sum(prod, axis=0, keepdims=True)
    dots_ref[...] = dots.reshape(BLK // 128, 128)


@jax.jit
def _tc_sweep(tt, ids_head):
    grid_spec = pltpu.PrefetchScalarGridSpec(
        num_scalar_prefetch=1,
        grid=(NBLK,),
        in_specs=[pl.BlockSpec((D, BLK), lambda i, s: (0, i))],
        out_specs=[
            pl.BlockSpec((BLK // 128, 128), lambda i, s: (i, 0)),
            pl.BlockSpec((D, 1), lambda i, s: (0, 0)),
            pl.BlockSpec((D, 1), lambda i, s: (0, 0)),
        ],
        scratch_shapes=[pltpu.VMEM((D, 1), jnp.float32)],
    )
    return pl.pallas_call(
        _tc_sweep_body,
        grid_spec=grid_spec,
        out_shape=[
            jax.ShapeDtypeStruct((DROWS, 128), jnp.float32),
            jax.ShapeDtypeStruct((D, 1), jnp.float32),
            jax.ShapeDtypeStruct((D, 1), jnp.float32),
        ],
    )(ids_head, tt)


def _dot16(rows_ref, base, cc):
    lane = lax.iota(jnp.int32, L)
    acc = jnp.zeros((L,), jnp.float32)
    for i in range(L):
        r = base + i
        p = rows_ref[r, pl.ds(0, L)] * cc[0]
        for c in range(1, len(cc)):
            p = p + rows_ref[r, pl.ds(c * L, L)] * cc[c]
        acc = jnp.where(lane == i, jnp.sum(p), acc)
    return acc


def _sc_body(dots, atable, w1t, b1, w2t, b2, idx2, act2, charc, objc,
             out_logit, out_act,
             idx_v, row_v, dest_v, char_v, obj_v,
             w1t_v, b1_v, w2t_v, b2_v,
             aidx_v, arows_v, lgt_v, aout_v,
             sem_rows, sem_act):
    w = lax.axis_index("s") * NC + lax.axis_index("c")
    lane = lax.iota(jnp.int32, L)

    pltpu.sync_copy(idx2.at[pl.ds(w * 4, 4)], idx_v)
    for k in range(4):
        for j in range(8):
            sl = pl.ds(j * L, L)
            row_v[k, sl] = lax.shift_right_logical(idx_v[k, sl], 7)
    row_cps = [
        pltpu.async_copy(
            dots.at[row_v.at[k]], dest_v.at[pl.ds(k * 128, 128)], sem_rows)
        for k in range(4)
    ]

    pltpu.sync_copy(act2.at[pl.ds(w, 1)], aidx_v)
    act_cp = pltpu.async_copy(atable.at[aidx_v.at[0]], arows_v, sem_act)

    pltpu.sync_copy(charc, char_v)
    pltpu.sync_copy(objc, obj_v)
    pltpu.sync_copy(w1t, w1t_v)
    pltpu.sync_copy(b1, b1_v)
    pltpu.sync_copy(w2t, w2t_v)
    pltpu.sync_copy(b2, b2_v)

    cat = ([char_v[pl.ds(k * L, L)] for k in range(CH)]
           + [obj_v[pl.ds(k * L, L)] for k in range(CH)])
    h = []
    for jg in range(CH):
        sl = pl.ds(jg * L, L)
        h.append(jnp.maximum(_dot16(w1t_v, jg * L, cat) + b1_v[sl], 0.0))
    ec = []
    for jg in range(CH):
        sl = pl.ds(jg * L, L)
        ec.append(_dot16(w2t_v, jg * L, h) + b2_v[sl])

    act_cp.wait()
    for g in range(APW // L):
        aout_v[pl.ds(g * L, L)] = _dot16(arows_v, g * L, ec)
    pltpu.sync_copy(aout_v, out_act.at[pl.ds(w * APW, APW)])

    for cp in row_cps:
        cp.wait()
    for g in range(GPW):
        ids16 = idx_v[g // 8, pl.ds((g % 8) * L, L)]
        lanes = ids16 & 127
        rows16 = jnp.full((L,), g * L, jnp.int32) + lane
        lgt_v[pl.ds(g * L, L)] = plsc.load_gather(dest_v, [rows16, lanes])
    pltpu.sync_copy(lgt_v, out_logit.at[pl.ds(w * RPW, RPW)])


@jax.jit
def _sc_gather(dots, atable, w1t, b1, w2t, b2, idx2, act2, charc, objc):
    mesh = plsc.VectorSubcoreMesh(
        core_axis_name="c", subcore_axis_name="s",
        num_cores=NC, num_subcores=NS,
    )
    call = functools.partial(
        pl.kernel,
        out_type=(
            jax.ShapeDtypeStruct((B,), jnp.float32),
            jax.ShapeDtypeStruct((A_PAD,), jnp.float32),
        ),
        mesh=mesh,
        compiler_params=pltpu.CompilerParams(
            needs_layout_passes=False, use_tc_tiling_on_sc=False),
        scratch_types=[
            pltpu.VMEM((4, 128), jnp.int32),
            pltpu.VMEM((4, 128), jnp.int32),
            pltpu.VMEM((RPW, 128), jnp.float32),
            pltpu.VMEM((D,), jnp.float32),
            pltpu.VMEM((D,), jnp.float32),
            pltpu.VMEM((D, 2 * D), jnp.float32),
            pltpu.VMEM((D,), jnp.float32),
            pltpu.VMEM((D, D), jnp.float32),
            pltpu.VMEM((D,), jnp.float32),
            pltpu.VMEM((1, APW), jnp.int32),
            pltpu.VMEM((APW, D), jnp.float32),
            pltpu.VMEM((RPW,), jnp.float32),
            pltpu.VMEM((APW,), jnp.float32),
            pltpu.SemaphoreType.DMA,
            pltpu.SemaphoreType.DMA,
        ],
    )(_sc_body)
    return call(dots, atable, w1t, b1, w2t, b2, idx2, act2, charc, objc)


def kernel(object_table, action_table, W1, b1, W2, b2, node_name_ids, action_ids):
    tt = object_table.T
    ids_head = node_name_ids[:1]
    dots, charc, objc = _tc_sweep(tt, ids_head)
    idx2 = node_name_ids.reshape(B // 128, 128)
    act2 = jnp.concatenate(
        [action_ids, jnp.zeros((A_PAD - action_ids.shape[0],), jnp.int32)]
    ).reshape(NW, APW)
    out_logit, out_act = _sc_gather(
        dots, action_table, W1.T, b1, W2.T, b2, idx2, act2,
        charc[:, 0], objc[:, 0])
    return jnp.concatenate([out_logit, out_act[: action_ids.shape[0]]])
